# baseline (speedup 1.0000x reference)
import functools

import jax
import jax.numpy as jnp
import numpy as np
from jax import lax
from jax.experimental import pallas as pl
from jax.experimental.pallas import tpu as pltpu
from jax.experimental.pallas import tpu_sc as plsc

_VOCAB = 100000
_D = 1024
_BATCH = 4
_SEQ = 2048
_N = _BATCH * _SEQ

_NC = 2
_NS = 16
_NW = _NC * _NS

_SCALE = float(np.sqrt(_D))


def _pe_table() -> np.ndarray:
    pos = np.arange(_SEQ, dtype=np.float64)[:, None]
    i = np.arange(0, _D, 2, dtype=np.float64)
    div = np.exp(-np.log(10000.0) * i / _D)
    pe = np.zeros((_SEQ, _D), dtype=np.float64)
    pe[:, 0::2] = np.sin(pos * div)
    pe[:, 1::2] = np.cos(pos * div)
    return pe.astype(np.float32)


_PE = _pe_table()


_POS_W = _SEQ // _NW
_FGR = 8
_FNS = _POS_W // _FGR
_GROWS = _BATCH * _FGR


def _sc_fused(table, tok, pe):
    mesh = plsc.VectorSubcoreMesh(core_axis_name="c", subcore_axis_name="s")

    @functools.partial(
        pl.kernel,
        mesh=mesh,
        out_type=jax.ShapeDtypeStruct((_N, _D), jnp.float32),
        scratch_types=[
            pltpu.VMEM((_BATCH * _POS_W,), jnp.int32),
        ] + [pltpu.VMEM((_GROWS, _D), jnp.float32) for _ in range(3)]
          + [pltpu.VMEM((_FGR, _D), jnp.float32) for _ in range(3)]
          + [pltpu.SemaphoreType.DMA for _ in range(7)],
    )
    def k(table_hbm, tok_hbm, pe_hbm, out_hbm, idx_v, *rest):
        gbufs = rest[0:3]
        pbufs = rest[3:6]
        gsems = rest[6:9]
        psems = rest[9:12]
        wsem = rest[12]
        wid = lax.axis_index("s") * _NC + lax.axis_index("c")
        pbase = wid * _POS_W
        for b in range(_BATCH):
            pltpu.sync_copy(tok_hbm.at[b, pl.ds(pbase, _POS_W)],
                            idx_v.at[pl.ds(b * _POS_W, _POS_W)])

        def _fire(s, m):
            for b in range(_BATCH):
                pltpu.async_copy(
                    table_hbm.at[idx_v.at[pl.ds(b * _POS_W + s * _FGR, _FGR)]],
                    gbufs[m].at[pl.ds(b * _FGR, _FGR)], gsems[m])
            pltpu.async_copy(
                pe_hbm.at[pl.ds(pbase + s * _FGR, _FGR)], pbufs[m], psems[m])

        def _step(s, m):
            g, p = gbufs[m], pbufs[m]
            pltpu.make_async_copy(
                table_hbm.at[pl.ds(0, _GROWS)], g, gsems[m]).wait()
            pltpu.make_async_copy(
                pe_hbm.at[pl.ds(0, _FGR)], p, psems[m]).wait()

            @pl.loop(0, _FGR)
            def _(r):
                for c in range(_D // 16):
                    cols = pl.ds(c * 16, 16)
                    pv = p.at[pl.ds(r, 1), cols][...]
                    gv = [g.at[pl.ds(r + 8 * b, 1), cols][...]
                          for b in range(_BATCH)]
                    for b in range(_BATCH):
                        g.at[pl.ds(r + 8 * b, 1), cols][...] = (
                            gv[b] * _SCALE + pv)

            @pl.when(s + 2 < _FNS)
            def _():
                @pl.when(s >= 1)
                def _():
                    pltpu.make_async_copy(
                        table_hbm.at[pl.ds(0, _GROWS)], g, wsem).wait()

                _fire(s + 2, (m + 2) % 3)

            for b in range(_BATCH):
                pltpu.async_copy(
                    g.at[pl.ds(b * _FGR, _FGR)],
                    out_hbm.at[pl.ds(b * _SEQ + pbase + s * _FGR, _FGR)],
                    wsem)

        _fire(0, 0)
        _fire(1, 1)

        @pl.loop(0, _FNS)
        def _(s):
            for mm in range(3):
                @pl.when(s % 3 == mm)
                def _(mm=mm):
                    _step(s, mm)

        for _ in range(3):
            pltpu.make_async_copy(
                table_hbm.at[pl.ds(0, _GROWS)], gbufs[0], wsem).wait()

    return k(table, tok, pe)


def kernel(tokens, table):
    out = _sc_fused(table, tokens.astype(jnp.int32), jnp.asarray(_PE))
    return out.reshape(_BATCH, _SEQ, _D)
